# baseline (speedup 1.0000x reference)
import functools

import jax
import jax.numpy as jnp
from jax.experimental import pallas as pl
from jax.experimental.pallas import tpu as pltpu
from jax.experimental.pallas import tpu_sc as plsc

_D = 768
_E = 8
_F = 768
_T = 2048
_BM = 256
_NH = 2
_TH = _T // _NH
_GH = _TH * 2 // _BM + _E
_PH = _GH * _BM


def _meta_body(x_ref, gw_ref, ltri_ref, pos1_ref, pos2_ref, w1_ref, w2_ref,
               gid_ref, mt_ref, nt_ref):
    th = x_ref.shape[0]
    x = x_ref[...]
    logits = jnp.dot(x, gw_ref[...], preferred_element_type=jnp.float32)
    e_iota = jax.lax.broadcasted_iota(jnp.int32, (th, _E), 1)
    idx1 = jnp.argmax(logits, axis=1)
    one1 = e_iota == idx1[:, None]
    m1 = jnp.max(logits, axis=1, keepdims=True)
    neg = jnp.finfo(jnp.float32).min
    l2 = jnp.where(one1, neg, logits)
    idx2 = jnp.argmax(l2, axis=1)
    one2 = e_iota == idx2[:, None]
    m2 = jnp.max(l2, axis=1, keepdims=True)
    w1 = 1.0 / (1.0 + jnp.exp(m2 - m1))
    w1_ref[...] = w1
    w2_ref[...] = 1.0 - w1

    sel = (one1 | one2).astype(jnp.float32)
    cnt = jnp.sum(sel, axis=0)
    tiles = jnp.floor((cnt + (_BM - 1)) / _BM)
    asz = tiles * _BM
    je = jax.lax.broadcasted_iota(jnp.int32, (_E, _E), 0)
    ee = jax.lax.broadcasted_iota(jnp.int32, (_E, _E), 1)
    lower = (je < ee).astype(jnp.float32)
    off = jnp.sum(lower * asz[:, None], axis=0)
    tprefix = jnp.sum(lower * tiles[:, None], axis=0)
    total = jnp.sum(tiles)

    csum = jnp.dot(ltri_ref[...], sel.astype(jnp.bfloat16).astype(jnp.float32),
                   preferred_element_type=jnp.float32)
    pos = off[None, :] + csum - 1.0
    pos1 = jnp.sum(jnp.where(one1, pos, 0.0), axis=1)
    pos2 = jnp.sum(jnp.where(one2, pos, 0.0), axis=1)
    pos1_ref[...] = pos1.astype(jnp.int32)
    pos2_ref[...] = pos2.astype(jnp.int32)

    gg = jax.lax.broadcasted_iota(jnp.int32, (_GH, _E), 0).astype(jnp.float32)
    gc = jnp.minimum(gg, total - 1.0)
    te = tprefix[None, :]
    act = (gc >= te) & (gc < te + tiles[None, :])
    actf = act.astype(jnp.float32)
    ef = jax.lax.broadcasted_iota(jnp.int32, (_GH, _E), 1).astype(jnp.float32)
    gid = jnp.sum(actf * ef, axis=1)
    mt = jnp.sum(actf * (off[None, :] / _BM + gc - te), axis=1)
    gid_ref[...] = gid.astype(jnp.int32)
    mt_ref[...] = mt.astype(jnp.int32)
    nt_ref[...] = jnp.full((8,), total, jnp.float32).astype(jnp.int32)


def _meta(x, gate_w):
    th = x.shape[0]
    t_iota = jax.lax.broadcasted_iota(jnp.int32, (th, th), 0)
    j_iota = jax.lax.broadcasted_iota(jnp.int32, (th, th), 1)
    ltri = (j_iota <= t_iota).astype(jnp.bfloat16)
    return pl.pallas_call(
        _meta_body,
        out_shape=[
            jax.ShapeDtypeStruct((th,), jnp.int32),
            jax.ShapeDtypeStruct((th,), jnp.int32),
            jax.ShapeDtypeStruct((th, 1), jnp.float32),
            jax.ShapeDtypeStruct((th, 1), jnp.float32),
            jax.ShapeDtypeStruct((_GH,), jnp.int32),
            jax.ShapeDtypeStruct((_GH,), jnp.int32),
            jax.ShapeDtypeStruct((8,), jnp.int32),
        ],
    )(x, gate_w, ltri)


def _sc_dispatch(x, pos1, pos2):
    info = plsc.get_sparse_core_info()
    nc, ns = info.num_cores, info.num_subcores
    nw = nc * ns
    th = x.shape[0]
    per = th // nw
    mesh = plsc.VectorSubcoreMesh(core_axis_name="c", subcore_axis_name="s")

    @functools.partial(
        pl.kernel, mesh=mesh,
        out_type=jax.ShapeDtypeStruct((_PH, _D), jnp.float32),
        scratch_types=[pltpu.VMEM((per,), jnp.int32),
                       pltpu.VMEM((per,), jnp.int32),
                       pltpu.VMEM((per, _D), jnp.float32),
                       pltpu.SemaphoreType.DMA],
    )
    def k(x_hbm, p1_hbm, p2_hbm, xs_hbm, idx1_v, idx2_v, row_v, sem):
        wid = jax.lax.axis_index("s") * nc + jax.lax.axis_index("c")
        base = wid * per
        pltpu.sync_copy(p1_hbm.at[pl.ds(base, per)], idx1_v)
        pltpu.sync_copy(p2_hbm.at[pl.ds(base, per)], idx2_v)
        pltpu.sync_copy(x_hbm.at[pl.ds(base, per)], row_v)
        d1 = pltpu.async_copy(row_v, xs_hbm.at[idx1_v], sem)
        d2 = pltpu.async_copy(row_v, xs_hbm.at[idx2_v], sem)
        d1.wait()
        d2.wait()

    return k(x, pos1, pos2)


def _sc_gather(y_sorted, pos1, pos2):
    info = plsc.get_sparse_core_info()
    nc, ns = info.num_cores, info.num_subcores
    nw = nc * ns
    th = pos1.shape[0]
    per = th // nw
    mesh = plsc.VectorSubcoreMesh(core_axis_name="c", subcore_axis_name="s")

    @functools.partial(
        pl.kernel, mesh=mesh,
        out_type=[jax.ShapeDtypeStruct((th, _D), jnp.float32),
                  jax.ShapeDtypeStruct((th, _D), jnp.float32)],
        scratch_types=[pltpu.VMEM((per,), jnp.int32),
                       pltpu.VMEM((per,), jnp.int32),
                       pltpu.VMEM((per, _D), jnp.float32),
                       pltpu.VMEM((per, _D), jnp.float32),
                       pltpu.SemaphoreType.DMA],
    )
    def k(ys_hbm, p1_hbm, p2_hbm, y1_hbm, y2_hbm,
          idx1_v, idx2_v, row1_v, row2_v, sem):
        wid = jax.lax.axis_index("s") * nc + jax.lax.axis_index("c")
        base = wid * per
        pltpu.sync_copy(p1_hbm.at[pl.ds(base, per)], idx1_v)
        pltpu.sync_copy(p2_hbm.at[pl.ds(base, per)], idx2_v)
        d1 = pltpu.async_copy(ys_hbm.at[idx1_v], row1_v, sem)
        d2 = pltpu.async_copy(ys_hbm.at[idx2_v], row2_v, sem)
        d1.wait()
        pltpu.sync_copy(row1_v, y1_hbm.at[pl.ds(base, per)])
        d2.wait()
        pltpu.sync_copy(row2_v, y2_hbm.at[pl.ds(base, per)])

    return k(y_sorted, pos1, pos2)


def _gmm_body(gid_ref, mt_ref, nt_ref, xs_ref, gu_ref, dn_ref, ys_ref):
    g = pl.program_id(0)

    @pl.when(g < nt_ref[0])
    def _():
        xb = xs_ref[...].astype(jnp.bfloat16)
        gu = gu_ref[0].astype(jnp.bfloat16)
        h = jnp.dot(xb, gu, preferred_element_type=jnp.float32)
        gate = h[:, :_F]
        up = h[:, _F:]
        actv = (gate * jax.lax.logistic(gate) * up).astype(jnp.bfloat16)
        dn = dn_ref[0].astype(jnp.bfloat16)
        ys_ref[...] = jnp.dot(actv, dn, preferred_element_type=jnp.float32)


def _gmm(x_sorted, gids, mtiles, nt, gate_up_proj, down_proj):
    grid_spec = pltpu.PrefetchScalarGridSpec(
        num_scalar_prefetch=3,
        grid=(_GH,),
        in_specs=[
            pl.BlockSpec((_BM, _D), lambda g, gid, mt, nt: (mt[g], 0)),
            pl.BlockSpec((1, _D, 2 * _F), lambda g, gid, mt, nt: (gid[g], 0, 0)),
            pl.BlockSpec((1, _F, _D), lambda g, gid, mt, nt: (gid[g], 0, 0)),
        ],
        out_specs=pl.BlockSpec((_BM, _D), lambda g, gid, mt, nt: (mt[g], 0)),
    )
    return pl.pallas_call(
        _gmm_body,
        grid_spec=grid_spec,
        out_shape=jax.ShapeDtypeStruct((_PH, _D), jnp.float32),
    )(gids, mtiles, nt, x_sorted, gate_up_proj, down_proj)


def _combine_body(y1_ref, y2_ref, w1_ref, w2_ref, out_ref):
    out_ref[...] = y1_ref[...] * w1_ref[...] + y2_ref[...] * w2_ref[...]


def _combine(y1, y2, w1, w2):
    th = y1.shape[0]
    return pl.pallas_call(
        _combine_body,
        grid=(2,),
        in_specs=[
            pl.BlockSpec((th // 2, _D), lambda i: (i, 0)),
            pl.BlockSpec((th // 2, _D), lambda i: (i, 0)),
            pl.BlockSpec((th // 2, 1), lambda i: (i, 0)),
            pl.BlockSpec((th // 2, 1), lambda i: (i, 0)),
        ],
        out_specs=pl.BlockSpec((th // 2, _D), lambda i: (i, 0)),
        out_shape=jax.ShapeDtypeStruct((th, _D), jnp.float32),
    )(y1, y2, w1, w2)


@jax.jit
def kernel(hidden_states, gate_w, gate_up_proj, down_proj):
    batch, seq, d = hidden_states.shape
    x = hidden_states.reshape(_T, d)

    metas = [_meta(x[h * _TH:(h + 1) * _TH], gate_w) for h in range(_NH)]
    xs = [_sc_dispatch(x[h * _TH:(h + 1) * _TH], m[0], m[1])
          for h, m in enumerate(metas)]
    ys = [_gmm(xs[h], m[4], m[5], m[6], gate_up_proj, down_proj)
          for h, m in enumerate(metas)]
    gat = [_sc_gather(ys[h], m[0], m[1]) for h, m in enumerate(metas)]
    outs = [_combine(gat[h][0], gat[h][1], m[2], m[3])
            for h, m in enumerate(metas)]

    return jnp.concatenate(outs, axis=0).reshape(batch, seq, d)


# speedup vs baseline: 1.6099x; 1.6099x over previous
import functools

import jax
import jax.numpy as jnp
from jax.experimental import pallas as pl
from jax.experimental.pallas import tpu as pltpu

_D_MODEL = 768
_N_EXPERTS = 8
_D_FF = 768


def _router_weights(x, gw):
    logits = jnp.dot(x, gw, preferred_element_type=jnp.float32)
    e_iota = jax.lax.broadcasted_iota(jnp.int32, logits.shape, 1)
    idx1 = jnp.argmax(logits, axis=1)
    one1 = e_iota == idx1[:, None]
    m1 = jnp.max(logits, axis=1, keepdims=True)
    neg = jnp.finfo(jnp.float32).min
    l2 = jnp.where(one1, neg, logits)
    idx2 = jnp.argmax(l2, axis=1)
    one2 = e_iota == idx2[:, None]
    m2 = jnp.max(l2, axis=1, keepdims=True)
    w1 = 1.0 / (1.0 + jnp.exp(m2 - m1))
    w2 = 1.0 - w1
    return jnp.where(one1, w1, jnp.where(one2, w2, 0.0))


def _moe_body(x_ref, gw_ref, gu_ref, dn_ref, out_ref, dr_ref):
    tb = pl.program_id(0)
    e = pl.program_id(1)

    @pl.when(e == 0)
    def _():
        dr_ref[...] = _router_weights(x_ref[...], gw_ref[...])

    x = x_ref[...].astype(jnp.bfloat16)
    gu = gu_ref[0].astype(jnp.bfloat16)
    h = jnp.dot(x, gu, preferred_element_type=jnp.float32)
    gate = h[:, :_D_FF]
    up = h[:, _D_FF:]
    act = (gate * jax.lax.logistic(gate) * up).astype(jnp.bfloat16)
    dn = dn_ref[0].astype(jnp.bfloat16)
    y = jnp.dot(act, dn, preferred_element_type=jnp.float32)
    dr = dr_ref[...]
    e_iota = jax.lax.broadcasted_iota(jnp.int32, dr.shape, 1)
    w = jnp.sum(jnp.where(e_iota == e, dr, 0.0), axis=1, keepdims=True)
    contrib = y * w

    @pl.when(e == 0)
    def _():
        out_ref[...] = contrib

    @pl.when(e > 0)
    def _():
        out_ref[...] += contrib


@jax.jit
def kernel(hidden_states, gate_w, gate_up_proj, down_proj):
    batch, seq, d = hidden_states.shape
    T = batch * seq
    x = hidden_states.reshape(T, d)

    TB = 1024
    out = pl.pallas_call(
        _moe_body,
        grid=(T // TB, _N_EXPERTS),
        in_specs=[
            pl.BlockSpec((TB, d), lambda tb, e: (tb, 0)),
            pl.BlockSpec((d, _N_EXPERTS), lambda tb, e: (0, 0)),
            pl.BlockSpec((1, d, 2 * _D_FF), lambda tb, e: (e, 0, 0)),
            pl.BlockSpec((1, _D_FF, d), lambda tb, e: (e, 0, 0)),
        ],
        out_specs=pl.BlockSpec((TB, d), lambda tb, e: (tb, 0)),
        out_shape=jax.ShapeDtypeStruct((T, d), jnp.float32),
        scratch_shapes=[pltpu.VMEM((TB, _N_EXPERTS), jnp.float32)],
    )(x, gate_w, gate_up_proj, down_proj)

    return out.reshape(batch, seq, d)
